# Initial kernel scaffold; baseline (speedup 1.0000x reference)
import jax
import jax.numpy as jnp
from jax.experimental import pallas as pl


def kernel(x_pmos, x_nmos, params, eidx_gate, eidx_sd, eidx_back):
    raise NotImplementedError("write your pallas kernel here")


# speedup vs baseline: 8.9457x; 8.9457x over previous
import functools

import jax
import jax.numpy as jnp
from jax import lax
from jax.experimental import pallas as pl
from jax.experimental.pallas import tpu as pltpu
from jax.experimental.pallas import tpu_sc as plsc

N = 50000
HALF = 25000
SPROWS = 25088
RPT = 1568
E = 800000
CH = 64
NCH = 782
EPT = NCH * CH
EPAD = 16 * EPT
W80 = 80
NEG = -1e30


def _gat_edges(h80, er1, esrc, edst):
    mesh = plsc.VectorSubcoreMesh(core_axis_name="c", subcore_axis_name="s")

    @functools.partial(
        pl.kernel,
        out_type=jax.ShapeDtypeStruct((2, SPROWS, W80), jnp.float32),
        mesh=mesh,
        compiler_params=pltpu.CompilerParams(
            use_tc_tiling_on_sc=False, needs_layout_passes=False),
        scratch_types=[
            pltpu.VMEM_SHARED((SPROWS, W80), jnp.float32),
            pltpu.VMEM((CH, W80), jnp.float32),
            pltpu.VMEM((CH,), jnp.float32),
            pltpu.VMEM((CH,), jnp.int32),
            pltpu.VMEM((CH,), jnp.int32),
            pltpu.SemaphoreType.DMA,
            pltpu.SemaphoreType.DMA,
        ],
    )
    def k(h_hbm, er_hbm, es_hbm, ed_hbm, out_hbm,
          acc_sp, gath, erv, isv, idv, sem1, sem2):
        c = lax.axis_index("c")
        s = lax.axis_index("s")
        lo = c * HALF
        hi = lo + HALF
        zero16 = jnp.zeros((16,), jnp.float32)

        @pl.loop(0, CH)
        def _(i):
            for q in range(W80 // 16):
                gath[i, pl.ds(q * 16, 16)] = zero16

        base = s * RPT
        off = 0
        for sz in (CH,) * 24 + (32,):
            pltpu.sync_copy(gath.at[pl.ds(0, sz)],
                            acc_sp.at[pl.ds(base + off, sz)])
            off += sz
        plsc.subcore_barrier()

        ebase = s * EPT

        @pl.loop(0, NCH)
        def _(kk):
            eoff = ebase + kk * CH
            pltpu.sync_copy(es_hbm.at[pl.ds(eoff, CH)], isv)
            pltpu.sync_copy(ed_hbm.at[pl.ds(eoff, CH)], idv)
            pltpu.async_copy(h_hbm.at[isv], gath, sem1).wait()
            pltpu.async_copy(er_hbm.at[idv], erv, sem2).wait()

            @pl.loop(0, CH, step=16)
            def _(j):
                d16 = idv[pl.ds(j, 16)]
                inr = (d16 >= lo) & (d16 < hi)
                erv[pl.ds(j, 16)] = jnp.where(inr, erv[pl.ds(j, 16)], NEG)
                idv[pl.ds(j, 16)] = jnp.clip(d16 - lo, 0, HALF)

            @pl.loop(0, CH)
            def _(i):
                i16 = jnp.full((16,), i, jnp.int32)
                er_b = plsc.load_gather(erv, [i16])
                e = gath[i, pl.ds(64, 16)] + er_b
                w = jnp.exp(jnp.maximum(e, 0.2 * e))
                for q in range(4):
                    gath[i, pl.ds(q * 16, 16)] = gath[i, pl.ds(q * 16, 16)] * w
                gath[i, pl.ds(64, 16)] = w

            pltpu.sync_copy(gath, acc_sp.at[idv], add=True)

        plsc.subcore_barrier()
        pltpu.sync_copy(acc_sp.at[pl.ds(base, RPT)],
                        out_hbm.at[c, pl.ds(base, RPT)])

    return k(h80, er1, esrc, edst)


def _proj_from_x(x, wbig, bias, widths):
    n, kdim = x.shape
    m = wbig.shape[1]
    b = 1000
    offs = [sum(widths[:t]) for t in range(len(widths))]

    def body(x_ref, w_ref, b_ref, *out_refs):
        y = jnp.dot(x_ref[...], w_ref[...],
                    preferred_element_type=jnp.float32) + b_ref[...]
        for r, o, wd in zip(out_refs, offs, widths):
            r[...] = y[:, o:o + wd]

    return pl.pallas_call(
        body,
        grid=(n // b,),
        in_specs=[
            pl.BlockSpec((b, kdim), lambda i: (i, 0)),
            pl.BlockSpec((kdim, m), lambda i: (0, 0)),
            pl.BlockSpec((1, m), lambda i: (0, 0)),
        ],
        out_specs=[pl.BlockSpec((b, wd), lambda i: (i, 0)) for wd in widths],
        out_shape=[jax.ShapeDtypeStruct((n, wd), jnp.float32) for wd in widths],
    )(x, wbig, bias)


def _proj_from_acc(accs, biases, wcat, widths):
    nacc = len(accs)
    m = wcat.shape[1]
    b = 1000
    nb = HALF // b
    offs = [sum(widths[:t]) for t in range(len(widths))]

    def body(*refs):
        acc_refs = refs[:nacc]
        b_refs = refs[nacc:2 * nacc]
        w_ref = refs[2 * nacc]
        out_refs = refs[2 * nacc + 1:]
        xs = None
        for a_ref, b_ref in zip(acc_refs, b_refs):
            blk = a_ref[...][0]
            den = blk[:, 64:65]
            h = jnp.where(den != 0.0, blk[:, :64] / den, 0.0) + b_ref[...]
            xs = h if xs is None else xs + h
        x = jnp.maximum(xs, 0.0)
        y = jnp.dot(x, w_ref[...], preferred_element_type=jnp.float32)
        for r, o, wd in zip(out_refs, offs, widths):
            r[...] = y[:, o:o + wd]

    in_specs = (
        [pl.BlockSpec((1, b, W80), lambda c, j: (c, j, 0))] * nacc
        + [pl.BlockSpec((1, 64), lambda c, j: (0, 0))] * nacc
        + [pl.BlockSpec((64, m), lambda c, j: (0, 0))]
    )
    return pl.pallas_call(
        body,
        grid=(2, nb),
        in_specs=in_specs,
        out_specs=[pl.BlockSpec((b, wd), lambda c, j: (c * nb + j, 0))
                   for wd in widths],
        out_shape=[jax.ShapeDtypeStruct((N, wd), jnp.float32) for wd in widths],
    )(*accs, *biases, wcat)


def _readout(acc_g, acc_sd, acc_bk, bg, bsd, bbk, w1t, b1, w2t, b2):
    b = 1000
    nb = HALF // b

    def body(ag, asd, abk, bg_r, bsd_r, bbk_r, w1, b1_r, w2, b2_r,
             out_ref, accum):
        c = pl.program_id(0)
        j = pl.program_id(1)

        @pl.when((c == 0) & (j == 0))
        def _():
            accum[...] = jnp.zeros_like(accum)

        def norm(ref):
            blk = ref[...][0]
            den = blk[:, 64:65]
            return jnp.where(den != 0.0, blk[:, :64] / den, 0.0)

        sp = jnp.sum(norm(asd) + norm(abk), axis=0, keepdims=True)
        sn = jnp.sum(norm(ag), axis=0, keepdims=True)
        accum[0:1, :] += sn
        accum[1:2, :] += sp

        @pl.when((c == 1) & (j == nb - 1))
        def _():
            mean_n = accum[0:1, :] / N + bg_r[...]
            mean_p = accum[1:2, :] / N + bsd_r[...] + bbk_r[...]
            z = 0.5 * (mean_p + mean_n)
            h1 = jnp.maximum(
                jnp.dot(z, w1[...], preferred_element_type=jnp.float32)
                + b1_r[...], 0.0)
            out_ref[...] = (jnp.dot(h1, w2[...],
                                    preferred_element_type=jnp.float32)
                            + b2_r[...])

    full = lambda c, j: (0, 0)
    return pl.pallas_call(
        body,
        grid=(2, nb),
        in_specs=(
            [pl.BlockSpec((1, b, W80), lambda c, j: (c, j, 0))] * 3
            + [pl.BlockSpec((1, 64), full)] * 3
            + [pl.BlockSpec((64, 64), full), pl.BlockSpec((1, 64), full)] * 2
        ),
        out_specs=pl.BlockSpec((1, 64), full),
        out_shape=jax.ShapeDtypeStruct((1, 64), jnp.float32),
        scratch_shapes=[pltpu.VMEM((8, 64), jnp.float32)],
    )(acc_g, acc_sd, acc_bk, bg, bsd, bbk, w1t, b1, w2t, b2)


def _rep16(v):
    return jnp.tile(v[:, None], (1, 16))


def _src_table_w(gp):
    wt = gp["W"].T
    return jnp.concatenate([wt, _rep16(wt @ gp["attn_l"])], axis=1)


def _er_w(gp):
    return _rep16(gp["W"].T @ gp["attn_r"])


def _pad_er(er16):
    return jnp.concatenate([er16[:, 0], jnp.full((1,), NEG, jnp.float32)])


def _prep_edges(eidx):
    es = eidx[0].astype(jnp.int32)
    ed = eidx[1].astype(jnp.int32)
    pad = EPAD - E
    es = jnp.concatenate([es, jnp.zeros((pad,), jnp.int32)])
    ed = jnp.concatenate([ed, jnp.full((pad,), N, jnp.int32)])
    return es, ed


def kernel(x_pmos, x_nmos, params, eidx_gate, eidx_sd, eidx_back):
    p = params
    esg, edg = _prep_edges(eidx_gate)
    ess, eds = _prep_edges(eidx_sd)
    esb, edb = _prep_edges(eidx_back)

    wcat_p1 = jnp.concatenate(
        [_src_table_w(p["l1_gate"]), _src_table_w(p["l1_back"]),
         _er_w(p["l1_sd"]), _er_w(p["l1_back"])], axis=1)
    wcat_n1 = jnp.concatenate(
        [_src_table_w(p["l1_sd"]), _er_w(p["l1_gate"])], axis=1)
    wbig_p1 = p["We_p"].T @ wcat_p1
    bias_p1 = (p["be_p"] @ wcat_p1)[None, :]
    wbig_n1 = p["We_n"].T @ wcat_n1
    bias_n1 = (p["be_n"] @ wcat_n1)[None, :]

    hg80, hb80, er_sd, er_bk = _proj_from_x(
        x_pmos, wbig_p1, bias_p1, [80, 80, 16, 16])
    hs80, er_g = _proj_from_x(x_nmos, wbig_n1, bias_n1, [80, 16])

    acc_g1 = _gat_edges(hg80, _pad_er(er_g), esg, edg)
    acc_s1 = _gat_edges(hs80, _pad_er(er_sd), ess, eds)
    acc_b1 = _gat_edges(hb80, _pad_er(er_bk), esb, edb)

    wcat_p2 = jnp.concatenate(
        [_src_table_w(p["l2_gate"]), _src_table_w(p["l2_back"]),
         _er_w(p["l2_sd"]), _er_w(p["l2_back"])], axis=1)
    wcat_n2 = jnp.concatenate(
        [_src_table_w(p["l2_sd"]), _er_w(p["l2_gate"])], axis=1)

    b_g1 = p["l1_gate"]["b"][None, :]
    b_s1 = p["l1_sd"]["b"][None, :]
    b_b1 = p["l1_back"]["b"][None, :]

    hg80_2, hb80_2, er_sd2, er_bk2 = _proj_from_acc(
        [acc_s1, acc_b1], [b_s1, b_b1], wcat_p2, [80, 80, 16, 16])
    hs80_2, er_g2 = _proj_from_acc([acc_g1], [b_g1], wcat_n2, [80, 16])

    acc_g2 = _gat_edges(hg80_2, _pad_er(er_g2), esg, edg)
    acc_s2 = _gat_edges(hs80_2, _pad_er(er_sd2), ess, eds)
    acc_b2 = _gat_edges(hb80_2, _pad_er(er_bk2), esb, edb)

    y = _readout(
        acc_g2, acc_s2, acc_b2,
        p["l2_gate"]["b"][None, :], p["l2_sd"]["b"][None, :],
        p["l2_back"]["b"][None, :],
        p["Wr1"].T, p["br1"][None, :], p["Wr2"].T, p["br2"][None, :])
    return y.reshape(-1)


# speedup vs baseline: 12.4017x; 1.3863x over previous
import functools

import jax
import jax.numpy as jnp
from jax import lax
from jax.experimental import pallas as pl
from jax.experimental.pallas import tpu as pltpu
from jax.experimental.pallas import tpu_sc as plsc

N = 50000
HALF = 25000
SPROWS = 25088
RPT = 1568
E = 800000
CH = 64
NCH = 782
EPT = NCH * CH
EPAD = 16 * EPT
W80 = 80
NEG = -1e30


def _gat_edges(h80, er1, esrc, edst):
    mesh = plsc.VectorSubcoreMesh(core_axis_name="c", subcore_axis_name="s")

    @functools.partial(
        pl.kernel,
        out_type=jax.ShapeDtypeStruct((2, SPROWS, W80), jnp.float32),
        mesh=mesh,
        compiler_params=pltpu.CompilerParams(
            use_tc_tiling_on_sc=False, needs_layout_passes=False),
        scratch_types=[
            pltpu.VMEM_SHARED((SPROWS, W80), jnp.float32),
            pltpu.VMEM((CH, W80), jnp.float32),
            pltpu.VMEM((CH,), jnp.float32),
            pltpu.VMEM((CH,), jnp.int32),
            pltpu.VMEM((CH,), jnp.int32),
            pltpu.SemaphoreType.DMA,
            pltpu.SemaphoreType.DMA,
        ],
    )
    def k(h_hbm, er_hbm, es_hbm, ed_hbm, out_hbm,
          acc_sp, gath, erv, isv, idv, sem1, sem2):
        c = lax.axis_index("c")
        s = lax.axis_index("s")
        lo = c * HALF
        hi = lo + HALF
        zero16 = jnp.zeros((16,), jnp.float32)

        @pl.loop(0, CH)
        def _(i):
            for q in range(W80 // 16):
                gath[i, pl.ds(q * 16, 16)] = zero16

        base = s * RPT
        off = 0
        for sz in (CH,) * 24 + (32,):
            pltpu.sync_copy(gath.at[pl.ds(0, sz)],
                            acc_sp.at[pl.ds(base + off, sz)])
            off += sz
        plsc.subcore_barrier()

        ebase = s * EPT

        @pl.loop(0, NCH)
        def _(kk):
            eoff = ebase + kk * CH
            cp_s = pltpu.async_copy(es_hbm.at[pl.ds(eoff, CH)], isv, sem1)
            cp_d = pltpu.async_copy(ed_hbm.at[pl.ds(eoff, CH)], idv, sem2)
            cp_s.wait()
            cp_d.wait()
            cp_h = pltpu.async_copy(h_hbm.at[isv], gath, sem1)
            cp_e = pltpu.async_copy(er_hbm.at[idv], erv, sem2)
            cp_h.wait()
            cp_e.wait()

            @pl.loop(0, CH, step=16)
            def _(j):
                d16 = idv[pl.ds(j, 16)]
                inr = (d16 >= lo) & (d16 < hi)
                erv[pl.ds(j, 16)] = jnp.where(inr, erv[pl.ds(j, 16)], NEG)
                idv[pl.ds(j, 16)] = jnp.clip(d16 - lo, 0, HALF)

            @pl.loop(0, CH)
            def _(i):
                i16 = jnp.full((16,), i, jnp.int32)
                er_b = plsc.load_gather(erv, [i16])
                e = gath[i, pl.ds(64, 16)] + er_b
                w = jnp.exp(jnp.maximum(e, 0.2 * e))
                for q in range(4):
                    gath[i, pl.ds(q * 16, 16)] = gath[i, pl.ds(q * 16, 16)] * w
                gath[i, pl.ds(64, 16)] = w

            pltpu.sync_copy(gath, acc_sp.at[idv], add=True)

        plsc.subcore_barrier()
        pltpu.sync_copy(acc_sp.at[pl.ds(base, RPT)],
                        out_hbm.at[c, pl.ds(base, RPT)])

    return k(h80, er1, esrc, edst)


def _proj_from_x(x, wbig, bias, widths):
    n, kdim = x.shape
    m = wbig.shape[1]
    b = 1000
    offs = [sum(widths[:t]) for t in range(len(widths))]

    def body(x_ref, w_ref, b_ref, *out_refs):
        y = jnp.dot(x_ref[...], w_ref[...],
                    preferred_element_type=jnp.float32) + b_ref[...]
        for r, o, wd in zip(out_refs, offs, widths):
            r[...] = y[:, o:o + wd]

    return pl.pallas_call(
        body,
        grid=(n // b,),
        in_specs=[
            pl.BlockSpec((b, kdim), lambda i: (i, 0)),
            pl.BlockSpec((kdim, m), lambda i: (0, 0)),
            pl.BlockSpec((1, m), lambda i: (0, 0)),
        ],
        out_specs=[pl.BlockSpec((b, wd), lambda i: (i, 0)) for wd in widths],
        out_shape=[jax.ShapeDtypeStruct((n, wd), jnp.float32) for wd in widths],
    )(x, wbig, bias)


def _proj_from_acc(accs, biases, wcat, widths):
    nacc = len(accs)
    m = wcat.shape[1]
    b = 1000
    nb = HALF // b
    offs = [sum(widths[:t]) for t in range(len(widths))]

    def body(*refs):
        acc_refs = refs[:nacc]
        b_refs = refs[nacc:2 * nacc]
        w_ref = refs[2 * nacc]
        out_refs = refs[2 * nacc + 1:]
        xs = None
        for a_ref, b_ref in zip(acc_refs, b_refs):
            blk = a_ref[...][0]
            den = blk[:, 64:65]
            h = jnp.where(den != 0.0, blk[:, :64] / den, 0.0) + b_ref[...]
            xs = h if xs is None else xs + h
        x = jnp.maximum(xs, 0.0)
        y = jnp.dot(x, w_ref[...], preferred_element_type=jnp.float32)
        for r, o, wd in zip(out_refs, offs, widths):
            r[...] = y[:, o:o + wd]

    in_specs = (
        [pl.BlockSpec((1, b, W80), lambda c, j: (c, j, 0))] * nacc
        + [pl.BlockSpec((1, 64), lambda c, j: (0, 0))] * nacc
        + [pl.BlockSpec((64, m), lambda c, j: (0, 0))]
    )
    return pl.pallas_call(
        body,
        grid=(2, nb),
        in_specs=in_specs,
        out_specs=[pl.BlockSpec((b, wd), lambda c, j: (c * nb + j, 0))
                   for wd in widths],
        out_shape=[jax.ShapeDtypeStruct((N, wd), jnp.float32) for wd in widths],
    )(*accs, *biases, wcat)


def _readout(acc_g, acc_sd, acc_bk, bg, bsd, bbk, w1t, b1, w2t, b2):
    b = 1000
    nb = HALF // b

    def body(ag, asd, abk, bg_r, bsd_r, bbk_r, w1, b1_r, w2, b2_r,
             out_ref, accum):
        c = pl.program_id(0)
        j = pl.program_id(1)

        @pl.when((c == 0) & (j == 0))
        def _():
            accum[...] = jnp.zeros_like(accum)

        def norm(ref):
            blk = ref[...][0]
            den = blk[:, 64:65]
            return jnp.where(den != 0.0, blk[:, :64] / den, 0.0)

        sp = jnp.sum(norm(asd) + norm(abk), axis=0, keepdims=True)
        sn = jnp.sum(norm(ag), axis=0, keepdims=True)
        accum[0:1, :] += sn
        accum[1:2, :] += sp

        @pl.when((c == 1) & (j == nb - 1))
        def _():
            mean_n = accum[0:1, :] / N + bg_r[...]
            mean_p = accum[1:2, :] / N + bsd_r[...] + bbk_r[...]
            z = 0.5 * (mean_p + mean_n)
            h1 = jnp.maximum(
                jnp.dot(z, w1[...], preferred_element_type=jnp.float32)
                + b1_r[...], 0.0)
            out_ref[...] = (jnp.dot(h1, w2[...],
                                    preferred_element_type=jnp.float32)
                            + b2_r[...])

    full = lambda c, j: (0, 0)
    return pl.pallas_call(
        body,
        grid=(2, nb),
        in_specs=(
            [pl.BlockSpec((1, b, W80), lambda c, j: (c, j, 0))] * 3
            + [pl.BlockSpec((1, 64), full)] * 3
            + [pl.BlockSpec((64, 64), full), pl.BlockSpec((1, 64), full)] * 2
        ),
        out_specs=pl.BlockSpec((1, 64), full),
        out_shape=jax.ShapeDtypeStruct((1, 64), jnp.float32),
        scratch_shapes=[pltpu.VMEM((8, 64), jnp.float32)],
    )(acc_g, acc_sd, acc_bk, bg, bsd, bbk, w1t, b1, w2t, b2)


def _rep16(v):
    return jnp.tile(v[:, None], (1, 16))


def _src_table_w(gp):
    wt = gp["W"].T
    return jnp.concatenate([wt, _rep16(wt @ gp["attn_l"])], axis=1)


def _er_w(gp):
    return _rep16(gp["W"].T @ gp["attn_r"])


def _pad_er(er16):
    return jnp.concatenate([er16[:, 0], jnp.full((1,), NEG, jnp.float32)])


def _prep_edges(eidx):
    es = eidx[0].astype(jnp.int32)
    ed = eidx[1].astype(jnp.int32)
    pad = EPAD - E
    es = jnp.concatenate([es, jnp.zeros((pad,), jnp.int32)])
    ed = jnp.concatenate([ed, jnp.full((pad,), N, jnp.int32)])
    return es, ed


def kernel(x_pmos, x_nmos, params, eidx_gate, eidx_sd, eidx_back):
    p = params
    esg, edg = _prep_edges(eidx_gate)
    ess, eds = _prep_edges(eidx_sd)
    esb, edb = _prep_edges(eidx_back)

    wcat_p1 = jnp.concatenate(
        [_src_table_w(p["l1_gate"]), _src_table_w(p["l1_back"]),
         _er_w(p["l1_sd"]), _er_w(p["l1_back"])], axis=1)
    wcat_n1 = jnp.concatenate(
        [_src_table_w(p["l1_sd"]), _er_w(p["l1_gate"])], axis=1)
    wbig_p1 = p["We_p"].T @ wcat_p1
    bias_p1 = (p["be_p"] @ wcat_p1)[None, :]
    wbig_n1 = p["We_n"].T @ wcat_n1
    bias_n1 = (p["be_n"] @ wcat_n1)[None, :]

    hg80, hb80, er_sd, er_bk = _proj_from_x(
        x_pmos, wbig_p1, bias_p1, [80, 80, 16, 16])
    hs80, er_g = _proj_from_x(x_nmos, wbig_n1, bias_n1, [80, 16])

    acc_g1 = _gat_edges(hg80, _pad_er(er_g), esg, edg)
    acc_s1 = _gat_edges(hs80, _pad_er(er_sd), ess, eds)
    acc_b1 = _gat_edges(hb80, _pad_er(er_bk), esb, edb)

    wcat_p2 = jnp.concatenate(
        [_src_table_w(p["l2_gate"]), _src_table_w(p["l2_back"]),
         _er_w(p["l2_sd"]), _er_w(p["l2_back"])], axis=1)
    wcat_n2 = jnp.concatenate(
        [_src_table_w(p["l2_sd"]), _er_w(p["l2_gate"])], axis=1)

    b_g1 = p["l1_gate"]["b"][None, :]
    b_s1 = p["l1_sd"]["b"][None, :]
    b_b1 = p["l1_back"]["b"][None, :]

    hg80_2, hb80_2, er_sd2, er_bk2 = _proj_from_acc(
        [acc_s1, acc_b1], [b_s1, b_b1], wcat_p2, [80, 80, 16, 16])
    hs80_2, er_g2 = _proj_from_acc([acc_g1], [b_g1], wcat_n2, [80, 16])

    acc_g2 = _gat_edges(hg80_2, _pad_er(er_g2), esg, edg)
    acc_s2 = _gat_edges(hs80_2, _pad_er(er_sd2), ess, eds)
    acc_b2 = _gat_edges(hb80_2, _pad_er(er_bk2), esb, edb)

    y = _readout(
        acc_g2, acc_s2, acc_b2,
        p["l2_gate"]["b"][None, :], p["l2_sd"]["b"][None, :],
        p["l2_back"]["b"][None, :],
        p["Wr1"].T, p["br1"][None, :], p["Wr2"].T, p["br2"][None, :])
    return y.reshape(-1)


# speedup vs baseline: 21.4609x; 1.7305x over previous
import functools

import jax
import jax.numpy as jnp
from jax import lax
from jax.experimental import pallas as pl
from jax.experimental.pallas import tpu as pltpu
from jax.experimental.pallas import tpu_sc as plsc

N = 50000
Q = 12500
E = 800000
EPAD = 800768
EPW = 25024
CHP = 64
NCHP = 391
REG = 25216
CH = 128
W80 = 80
NEG = -1e30

_MESH = dict(core_axis_name="c", subcore_axis_name="s")
_CP = dict(use_tc_tiling_on_sc=False, needs_layout_passes=False)


def _partition_edges(esrc, edst):
    @functools.partial(
        pl.kernel,
        out_type=(jax.ShapeDtypeStruct((4, 32, REG), jnp.int32),
                  jax.ShapeDtypeStruct((4, 32, REG), jnp.int32),
                  jax.ShapeDtypeStruct((32, 8), jnp.int32)),
        mesh=plsc.VectorSubcoreMesh(**_MESH),
        compiler_params=pltpu.CompilerParams(**_CP),
        scratch_types=[
            pltpu.VMEM((CHP,), jnp.int32),
            pltpu.VMEM((CHP,), jnp.int32),
            pltpu.VMEM((4, 192), jnp.int32),
            pltpu.VMEM((4, 192), jnp.int32),
            pltpu.VMEM((16,), jnp.int32),
            pltpu.SemaphoreType.DMA,
            pltpu.SemaphoreType.DMA,
        ],
    )
    def k(es_hbm, ed_hbm, ps_hbm, pd_hbm, cnt_hbm,
          isv, idv, stg_s, stg_d, cntv, sem1, sem2):
        c = lax.axis_index("c")
        s = lax.axis_index("s")
        wid = c * 16 + s
        ebase = wid * EPW
        lane = lax.iota(jnp.int32, 16)

        def chunk_body(t, carry):
            curs = list(carry[:4])
            outs = list(carry[4:])
            eoff = ebase + t * CHP
            cp1 = pltpu.async_copy(es_hbm.at[pl.ds(eoff, CHP)], isv, sem1)
            cp2 = pltpu.async_copy(ed_hbm.at[pl.ds(eoff, CHP)], idv, sem2)
            cp1.wait()
            cp2.wait()
            for g in range(CHP // 16):
                d16 = idv[pl.ds(g * 16, 16)]
                s16 = isv[pl.ds(g * 16, 16)]
                masks = [d16 < Q,
                         (d16 >= Q) & (d16 < 2 * Q),
                         (d16 >= 2 * Q) & (d16 < 3 * Q),
                         d16 >= 3 * Q]
                for b in range(4):
                    mb = masks[b]
                    plsc.store_compressed(stg_s.at[b, pl.ds(curs[b], 16)],
                                          s16, mask=mb)
                    plsc.store_compressed(stg_d.at[b, pl.ds(curs[b], 16)],
                                          d16, mask=mb)
                    newc = curs[b] + jnp.sum(jnp.where(mb, 1, 0))
                    fl = newc >= 64

                    @pl.when(fl)
                    def _(b=b, ob=pl.multiple_of(outs[b], 64)):
                        pltpu.sync_copy(stg_s.at[b, pl.ds(0, 64)],
                                        ps_hbm.at[b, wid, pl.ds(ob, 64)])
                        pltpu.sync_copy(stg_d.at[b, pl.ds(0, 64)],
                                        pd_hbm.at[b, wid, pl.ds(ob, 64)])
                        stg_s[b, pl.ds(0, 16)] = stg_s[b, pl.ds(64, 16)]
                        stg_d[b, pl.ds(0, 16)] = stg_d[b, pl.ds(64, 16)]

                    curs[b] = newc - jnp.where(fl, 64, 0)
                    outs[b] = outs[b] + jnp.where(fl, 64, 0)
            return (*curs, *outs)

        zero = jnp.int32(0)
        carry = lax.fori_loop(0, NCHP, chunk_body, (zero,) * 8)
        curs = carry[:4]
        outs = [pl.multiple_of(o, 64) for o in carry[4:]]
        pad_s = jnp.zeros((16,), jnp.int32)
        pad_d = jnp.full((16,), N, jnp.int32)
        cnts = []
        for b in range(4):
            for kk in range(4):
                stg_s[b, pl.ds(curs[b] + kk * 16, 16)] = pad_s
                stg_d[b, pl.ds(curs[b] + kk * 16, 16)] = pad_d
            pltpu.sync_copy(stg_s.at[b, pl.ds(0, 64)],
                            ps_hbm.at[b, wid, pl.ds(outs[b], 64)])
            pltpu.sync_copy(stg_d.at[b, pl.ds(0, 64)],
                            pd_hbm.at[b, wid, pl.ds(outs[b], 64)])
            for kk in range(8):
                stg_s[b, pl.ds(kk * 16, 16)] = pad_s
                stg_d[b, pl.ds(kk * 16, 16)] = pad_d
            pltpu.sync_copy(stg_s.at[b, pl.ds(0, 64)],
                            ps_hbm.at[b, wid, pl.ds(outs[b] + 64, 64)])
            pltpu.sync_copy(stg_d.at[b, pl.ds(0, 64)],
                            pd_hbm.at[b, wid, pl.ds(outs[b] + 64, 64)])
            pltpu.sync_copy(stg_s.at[b, pl.ds(64, 64)],
                            ps_hbm.at[b, wid, pl.ds(outs[b] + 128, 64)])
            pltpu.sync_copy(stg_d.at[b, pl.ds(64, 64)],
                            pd_hbm.at[b, wid, pl.ds(outs[b] + 128, 64)])
            cnts.append(outs[b] + curs[b])
        cv = jnp.where(lane == 0, cnts[0],
                       jnp.where(lane == 1, cnts[1],
                                 jnp.where(lane == 2, cnts[2],
                                           jnp.where(lane == 3, cnts[3], 0))))
        cntv[pl.ds(0, 16)] = cv
        pltpu.sync_copy(cntv.at[pl.ds(0, 8)], cnt_hbm.at[wid, pl.ds(0, 8)])

    return k(esrc, edst)


def _gat_edges(h80, er1, psrc, pdst, counts):
    @functools.partial(
        pl.kernel,
        out_type=jax.ShapeDtypeStruct((4, Q, W80), jnp.float32),
        mesh=plsc.VectorSubcoreMesh(**_MESH),
        compiler_params=pltpu.CompilerParams(**_CP),
        scratch_types=[
            pltpu.VMEM_SHARED((Q, W80), jnp.float32),
            pltpu.VMEM((CH, W80), jnp.float32),
            pltpu.VMEM((CH,), jnp.float32),
            pltpu.VMEM((CH,), jnp.int32),
            pltpu.VMEM((CH,), jnp.int32),
            pltpu.VMEM((CH,), jnp.int32),
            pltpu.VMEM((16,), jnp.int32),
            pltpu.SemaphoreType.DMA,
            pltpu.SemaphoreType.DMA,
        ],
    )
    def k(h_hbm, er_hbm, ps_hbm, pd_hbm, cnt_hbm, out_hbm,
          acc_sp, gath, erv, isv, idv, ldv, cntv, sem1, sem2):
        c = lax.axis_index("c")
        s = lax.axis_index("s")
        lane = lax.iota(jnp.int32, 16)
        zero16 = jnp.zeros((16,), jnp.float32)
        base = pl.multiple_of(s * 784, 16)

        for p in range(2):
            b = 2 * c + p
            lo = b * Q

            @pl.loop(0, CH)
            def _(i):
                for q5 in range(W80 // 16):
                    gath[i, pl.ds(q5 * 16, 16)] = zero16

            @pl.when(s < 15)
            def _():
                off = 0
                for sz in (CH,) * 6 + (16,):
                    pltpu.sync_copy(gath.at[pl.ds(0, sz)],
                                    acc_sp.at[pl.ds(base + off, sz)])
                    off += sz

            @pl.when(s == 15)
            def _():
                off = 0
                for sz in (CH,) * 5 + (100,):
                    pltpu.sync_copy(gath.at[pl.ds(0, sz)],
                                    acc_sp.at[pl.ds(11760 + off, sz)])
                    off += sz

            plsc.subcore_barrier()

            for r in (s, s + 16):
                pltpu.sync_copy(cnt_hbm.at[r, pl.ds(0, 8)],
                                cntv.at[pl.ds(0, 8)])
                cv = cntv[pl.ds(0, 16)]
                n = jnp.sum(jnp.where(lane == b, cv, 0))
                trips = lax.shift_right_logical(n + (CH - 1), 7)

                def chunk(t, _, b=b, r=r, lo=lo):
                    eoff = pl.multiple_of(t * CH, CH)
                    cp1 = pltpu.async_copy(ps_hbm.at[b, r, pl.ds(eoff, CH)],
                                           isv, sem1)
                    cp2 = pltpu.async_copy(pd_hbm.at[b, r, pl.ds(eoff, CH)],
                                           idv, sem2)
                    cp1.wait()
                    cp2.wait()
                    cp3 = pltpu.async_copy(h_hbm.at[isv], gath, sem1)
                    cp4 = pltpu.async_copy(er_hbm.at[idv], erv, sem2)
                    cp3.wait()
                    cp4.wait()

                    @pl.loop(0, CH, step=16)
                    def _(j):
                        d16 = idv[pl.ds(j, 16)]
                        ldv[pl.ds(j, 16)] = jnp.clip(d16 - lo, 0, Q - 1)

                    @pl.loop(0, CH)
                    def _(i):
                        i16 = jnp.full((16,), i, jnp.int32)
                        er_b = plsc.load_gather(erv, [i16])
                        e = gath[i, pl.ds(64, 16)] + er_b
                        w = jnp.exp(jnp.maximum(e, 0.2 * e))
                        for q5 in range(4):
                            gath[i, pl.ds(q5 * 16, 16)] = (
                                gath[i, pl.ds(q5 * 16, 16)] * w)
                        gath[i, pl.ds(64, 16)] = w

                    pltpu.sync_copy(gath, acc_sp.at[ldv], add=True)
                    return 0

                lax.fori_loop(0, trips, chunk, jnp.int32(0))

            plsc.subcore_barrier()

            @pl.when(s < 15)
            def _(b=b):
                pltpu.sync_copy(acc_sp.at[pl.ds(base, 784)],
                                out_hbm.at[b, pl.ds(base, 784)])

            @pl.when(s == 15)
            def _(b=b):
                pltpu.sync_copy(acc_sp.at[pl.ds(11760, 740)],
                                out_hbm.at[b, pl.ds(11760, 740)])

    out = k(h80, er1, psrc, pdst, counts)
    return out.reshape(N, W80)


def _proj_from_x(x, wbig, bias, widths):
    n, kdim = x.shape
    m = wbig.shape[1]
    bsz = 1000
    offs = [sum(widths[:t]) for t in range(len(widths))]

    def body(x_ref, w_ref, b_ref, *out_refs):
        y = jnp.dot(x_ref[...], w_ref[...],
                    preferred_element_type=jnp.float32) + b_ref[...]
        for r, o, wd in zip(out_refs, offs, widths):
            r[...] = y[:, o:o + wd]

    return pl.pallas_call(
        body,
        grid=(n // bsz,),
        in_specs=[
            pl.BlockSpec((bsz, kdim), lambda i: (i, 0)),
            pl.BlockSpec((kdim, m), lambda i: (0, 0)),
            pl.BlockSpec((1, m), lambda i: (0, 0)),
        ],
        out_specs=[pl.BlockSpec((bsz, wd), lambda i: (i, 0)) for wd in widths],
        out_shape=[jax.ShapeDtypeStruct((n, wd), jnp.float32) for wd in widths],
    )(x, wbig, bias)


def _proj_from_acc(accs, biases, wcat, widths):
    nacc = len(accs)
    m = wcat.shape[1]
    bsz = 1000
    offs = [sum(widths[:t]) for t in range(len(widths))]

    def body(*refs):
        acc_refs = refs[:nacc]
        b_refs = refs[nacc:2 * nacc]
        w_ref = refs[2 * nacc]
        out_refs = refs[2 * nacc + 1:]
        xs = None
        for a_ref, b_ref in zip(acc_refs, b_refs):
            blk = a_ref[...]
            den = blk[:, 64:65]
            h = jnp.where(den != 0.0, blk[:, :64] / den, 0.0) + b_ref[...]
            xs = h if xs is None else xs + h
        x = jnp.maximum(xs, 0.0)
        y = jnp.dot(x, w_ref[...], preferred_element_type=jnp.float32)
        for r, o, wd in zip(out_refs, offs, widths):
            r[...] = y[:, o:o + wd]

    in_specs = (
        [pl.BlockSpec((bsz, W80), lambda i: (i, 0))] * nacc
        + [pl.BlockSpec((1, 64), lambda i: (0, 0))] * nacc
        + [pl.BlockSpec((64, m), lambda i: (0, 0))]
    )
    return pl.pallas_call(
        body,
        grid=(N // bsz,),
        in_specs=in_specs,
        out_specs=[pl.BlockSpec((bsz, wd), lambda i: (i, 0))
                   for wd in widths],
        out_shape=[jax.ShapeDtypeStruct((N, wd), jnp.float32) for wd in widths],
    )(*accs, *biases, wcat)


def _readout(acc_g, acc_sd, acc_bk, bg, bsd, bbk, w1t, b1, w2t, b2):
    bsz = 1000
    nb = N // bsz

    def body(ag, asd, abk, bg_r, bsd_r, bbk_r, w1, b1_r, w2, b2_r,
             out_ref, accum):
        i = pl.program_id(0)

        @pl.when(i == 0)
        def _():
            accum[...] = jnp.zeros_like(accum)

        def norm(ref):
            blk = ref[...]
            den = blk[:, 64:65]
            return jnp.where(den != 0.0, blk[:, :64] / den, 0.0)

        accum[0:1, :] += jnp.sum(norm(ag), axis=0, keepdims=True)
        accum[1:2, :] += jnp.sum(norm(asd) + norm(abk), axis=0, keepdims=True)

        @pl.when(i == nb - 1)
        def _():
            mean_n = accum[0:1, :] / N + bg_r[...]
            mean_p = accum[1:2, :] / N + bsd_r[...] + bbk_r[...]
            z = 0.5 * (mean_p + mean_n)
            h1 = jnp.maximum(
                jnp.dot(z, w1[...], preferred_element_type=jnp.float32)
                + b1_r[...], 0.0)
            out_ref[...] = (jnp.dot(h1, w2[...],
                                    preferred_element_type=jnp.float32)
                            + b2_r[...])

    full = lambda i: (0, 0)
    return pl.pallas_call(
        body,
        grid=(nb,),
        in_specs=(
            [pl.BlockSpec((bsz, W80), lambda i: (i, 0))] * 3
            + [pl.BlockSpec((1, 64), full)] * 3
            + [pl.BlockSpec((64, 64), full), pl.BlockSpec((1, 64), full)] * 2
        ),
        out_specs=pl.BlockSpec((1, 64), full),
        out_shape=jax.ShapeDtypeStruct((1, 64), jnp.float32),
        scratch_shapes=[pltpu.VMEM((8, 64), jnp.float32)],
    )(acc_g, acc_sd, acc_bk, bg, bsd, bbk, w1t, b1, w2t, b2)


def _rep16(v):
    return jnp.tile(v[:, None], (1, 16))


def _src_table_w(gp):
    wt = gp["W"].T
    return jnp.concatenate([wt, _rep16(wt @ gp["attn_l"])], axis=1)


def _er_w(gp):
    return _rep16(gp["W"].T @ gp["attn_r"])


def _pad_er(er16):
    return jnp.concatenate([er16[:, 0], jnp.full((1,), NEG, jnp.float32)])


def _prep_edges(eidx):
    es = eidx[0].astype(jnp.int32)
    ed = eidx[1].astype(jnp.int32)
    pad = EPAD - E
    es = jnp.concatenate([es, jnp.zeros((pad,), jnp.int32)])
    ed = jnp.concatenate([ed, jnp.full((pad,), N, jnp.int32)])
    return es, ed


def kernel(x_pmos, x_nmos, params, eidx_gate, eidx_sd, eidx_back):
    p = params
    esg, edg = _prep_edges(eidx_gate)
    ess, eds = _prep_edges(eidx_sd)
    esb, edb = _prep_edges(eidx_back)

    part_g = _partition_edges(esg, edg)
    part_s = _partition_edges(ess, eds)
    part_b = _partition_edges(esb, edb)

    wcat_p1 = jnp.concatenate(
        [_src_table_w(p["l1_gate"]), _src_table_w(p["l1_back"]),
         _er_w(p["l1_sd"]), _er_w(p["l1_back"])], axis=1)
    wcat_n1 = jnp.concatenate(
        [_src_table_w(p["l1_sd"]), _er_w(p["l1_gate"])], axis=1)
    wbig_p1 = p["We_p"].T @ wcat_p1
    bias_p1 = (p["be_p"] @ wcat_p1)[None, :]
    wbig_n1 = p["We_n"].T @ wcat_n1
    bias_n1 = (p["be_n"] @ wcat_n1)[None, :]

    hg80, hb80, er_sd, er_bk = _proj_from_x(
        x_pmos, wbig_p1, bias_p1, [80, 80, 16, 16])
    hs80, er_g = _proj_from_x(x_nmos, wbig_n1, bias_n1, [80, 16])

    acc_g1 = _gat_edges(hg80, _pad_er(er_g), *part_g)
    acc_s1 = _gat_edges(hs80, _pad_er(er_sd), *part_s)
    acc_b1 = _gat_edges(hb80, _pad_er(er_bk), *part_b)

    wcat_p2 = jnp.concatenate(
        [_src_table_w(p["l2_gate"]), _src_table_w(p["l2_back"]),
         _er_w(p["l2_sd"]), _er_w(p["l2_back"])], axis=1)
    wcat_n2 = jnp.concatenate(
        [_src_table_w(p["l2_sd"]), _er_w(p["l2_gate"])], axis=1)

    b_g1 = p["l1_gate"]["b"][None, :]
    b_s1 = p["l1_sd"]["b"][None, :]
    b_b1 = p["l1_back"]["b"][None, :]

    hg80_2, hb80_2, er_sd2, er_bk2 = _proj_from_acc(
        [acc_s1, acc_b1], [b_s1, b_b1], wcat_p2, [80, 80, 16, 16])
    hs80_2, er_g2 = _proj_from_acc([acc_g1], [b_g1], wcat_n2, [80, 16])

    acc_g2 = _gat_edges(hg80_2, _pad_er(er_g2), *part_g)
    acc_s2 = _gat_edges(hs80_2, _pad_er(er_sd2), *part_s)
    acc_b2 = _gat_edges(hb80_2, _pad_er(er_bk2), *part_b)

    y = _readout(
        acc_g2, acc_s2, acc_b2,
        p["l2_gate"]["b"][None, :], p["l2_sd"]["b"][None, :],
        p["l2_back"]["b"][None, :],
        p["Wr1"].T, p["br1"][None, :], p["Wr2"].T, p["br2"][None, :])
    return y.reshape(-1)


# speedup vs baseline: 28.5869x; 1.3320x over previous
import functools

import jax
import jax.numpy as jnp
from jax import lax
from jax.experimental import pallas as pl
from jax.experimental.pallas import tpu as pltpu
from jax.experimental.pallas import tpu_sc as plsc

N = 50000
Q = 12500
E = 800000
EPAD = 800768
EPW = 25024
CHP = 64
NCHP = 391
REG = 25216
CH = 128
W80 = 80
NEG = -1e30

_MESH = dict(core_axis_name="c", subcore_axis_name="s")
_CP = dict(use_tc_tiling_on_sc=False, needs_layout_passes=False)


def _partition_edges(esrc, edst):
    @functools.partial(
        pl.kernel,
        out_type=(jax.ShapeDtypeStruct((4, 32, REG), jnp.int32),
                  jax.ShapeDtypeStruct((4, 32, REG), jnp.int32),
                  jax.ShapeDtypeStruct((32, 8), jnp.int32)),
        mesh=plsc.VectorSubcoreMesh(**_MESH),
        compiler_params=pltpu.CompilerParams(**_CP),
        scratch_types=[
            pltpu.VMEM((CHP,), jnp.int32),
            pltpu.VMEM((CHP,), jnp.int32),
            pltpu.VMEM((CHP,), jnp.int32),
            pltpu.VMEM((CHP,), jnp.int32),
            pltpu.VMEM((4, 192), jnp.int32),
            pltpu.VMEM((4, 192), jnp.int32),
            pltpu.VMEM((16,), jnp.int32),
            pltpu.SemaphoreType.DMA,
            pltpu.SemaphoreType.DMA,
            pltpu.SemaphoreType.DMA,
            pltpu.SemaphoreType.DMA,
        ],
    )
    def k(es_hbm, ed_hbm, ps_hbm, pd_hbm, cnt_hbm,
          isvA, idvA, isvB, idvB, stg_s, stg_d, cntv,
          semSA, semDA, semSB, semDB):
        c = lax.axis_index("c")
        s = lax.axis_index("s")
        wid = c * 16 + s
        ebase = wid * EPW
        lane = lax.iota(jnp.int32, 16)

        def issue(t, iv, dv, s1, s2):
            eoff = ebase + t * CHP
            pltpu.async_copy(es_hbm.at[pl.ds(eoff, CHP)], iv, s1)
            pltpu.async_copy(ed_hbm.at[pl.ds(eoff, CHP)], dv, s2)

        def wait_idx(t, iv, dv, s1, s2):
            eoff = ebase + t * CHP
            pltpu.make_async_copy(es_hbm.at[pl.ds(eoff, CHP)], iv, s1).wait()
            pltpu.make_async_copy(ed_hbm.at[pl.ds(eoff, CHP)], dv, s2).wait()

        def process(isv, idv, carry):
            curs = list(carry[:4])
            outs = list(carry[4:])
            for g in range(CHP // 16):
                d16 = idv[pl.ds(g * 16, 16)]
                s16 = isv[pl.ds(g * 16, 16)]
                masks = [d16 < Q,
                         (d16 >= Q) & (d16 < 2 * Q),
                         (d16 >= 2 * Q) & (d16 < 3 * Q),
                         d16 >= 3 * Q]
                for b in range(4):
                    mb = masks[b]
                    plsc.store_compressed(stg_s.at[b, pl.ds(curs[b], 16)],
                                          s16, mask=mb)
                    plsc.store_compressed(stg_d.at[b, pl.ds(curs[b], 16)],
                                          d16, mask=mb)
                    newc = curs[b] + jnp.sum(jnp.where(mb, 1, 0))
                    fl = newc >= 64

                    @pl.when(fl)
                    def _(b=b, ob=pl.multiple_of(outs[b], 64)):
                        pltpu.sync_copy(stg_s.at[b, pl.ds(0, 64)],
                                        ps_hbm.at[b, wid, pl.ds(ob, 64)])
                        pltpu.sync_copy(stg_d.at[b, pl.ds(0, 64)],
                                        pd_hbm.at[b, wid, pl.ds(ob, 64)])
                        stg_s[b, pl.ds(0, 16)] = stg_s[b, pl.ds(64, 16)]
                        stg_d[b, pl.ds(0, 16)] = stg_d[b, pl.ds(64, 16)]

                    curs[b] = newc - jnp.where(fl, 64, 0)
                    outs[b] = outs[b] + jnp.where(fl, 64, 0)
            return (*curs, *outs)

        def pair_body(u, carry):
            t0 = u * 2
            issue(t0 + 1, isvB, idvB, semSB, semDB)
            wait_idx(t0, isvA, idvA, semSA, semDA)
            carry = process(isvA, idvA, carry)
            issue(t0 + 2, isvA, idvA, semSA, semDA)
            wait_idx(t0 + 1, isvB, idvB, semSB, semDB)
            carry = process(isvB, idvB, carry)
            return carry

        zero = jnp.int32(0)
        issue(jnp.int32(0), isvA, idvA, semSA, semDA)
        carry = lax.fori_loop(0, (NCHP - 1) // 2, pair_body, (zero,) * 8)
        wait_idx(jnp.int32(NCHP - 1), isvA, idvA, semSA, semDA)
        carry = process(isvA, idvA, carry)
        curs = carry[:4]
        outs = [pl.multiple_of(o, 64) for o in carry[4:]]
        pad_s = jnp.zeros((16,), jnp.int32)
        pad_d = jnp.full((16,), N, jnp.int32)
        cnts = []
        for b in range(4):
            for kk in range(4):
                stg_s[b, pl.ds(curs[b] + kk * 16, 16)] = pad_s
                stg_d[b, pl.ds(curs[b] + kk * 16, 16)] = pad_d
            pltpu.sync_copy(stg_s.at[b, pl.ds(0, 64)],
                            ps_hbm.at[b, wid, pl.ds(outs[b], 64)])
            pltpu.sync_copy(stg_d.at[b, pl.ds(0, 64)],
                            pd_hbm.at[b, wid, pl.ds(outs[b], 64)])
            for kk in range(8):
                stg_s[b, pl.ds(kk * 16, 16)] = pad_s
                stg_d[b, pl.ds(kk * 16, 16)] = pad_d
            pltpu.sync_copy(stg_s.at[b, pl.ds(0, 64)],
                            ps_hbm.at[b, wid, pl.ds(outs[b] + 64, 64)])
            pltpu.sync_copy(stg_d.at[b, pl.ds(0, 64)],
                            pd_hbm.at[b, wid, pl.ds(outs[b] + 64, 64)])
            pltpu.sync_copy(stg_s.at[b, pl.ds(64, 64)],
                            ps_hbm.at[b, wid, pl.ds(outs[b] + 128, 64)])
            pltpu.sync_copy(stg_d.at[b, pl.ds(64, 64)],
                            pd_hbm.at[b, wid, pl.ds(outs[b] + 128, 64)])
            cnts.append(outs[b] + curs[b])
        cv = jnp.where(lane == 0, cnts[0],
                       jnp.where(lane == 1, cnts[1],
                                 jnp.where(lane == 2, cnts[2],
                                           jnp.where(lane == 3, cnts[3], 0))))
        cntv[pl.ds(0, 16)] = cv
        pltpu.sync_copy(cntv.at[pl.ds(0, 8)], cnt_hbm.at[wid, pl.ds(0, 8)])

    return k(esrc, edst)


def _gat_edges(h80, er1, psrc, pdst, counts):
    @functools.partial(
        pl.kernel,
        out_type=jax.ShapeDtypeStruct((4, Q, W80), jnp.float32),
        mesh=plsc.VectorSubcoreMesh(**_MESH),
        compiler_params=pltpu.CompilerParams(**_CP),
        scratch_types=[
            pltpu.VMEM_SHARED((Q, W80), jnp.float32),
            pltpu.VMEM((CH, W80), jnp.float32),
            pltpu.VMEM((CH, W80), jnp.float32),
            pltpu.VMEM((CH,), jnp.float32),
            pltpu.VMEM((CH,), jnp.float32),
            pltpu.VMEM((CH,), jnp.int32),
            pltpu.VMEM((CH,), jnp.int32),
            pltpu.VMEM((CH,), jnp.int32),
            pltpu.VMEM((CH,), jnp.int32),
            pltpu.VMEM((CH,), jnp.int32),
            pltpu.VMEM((16,), jnp.int32),
            pltpu.SemaphoreType.DMA,
            pltpu.SemaphoreType.DMA,
            pltpu.SemaphoreType.DMA,
            pltpu.SemaphoreType.DMA,
        ],
    )
    def k(h_hbm, er_hbm, ps_hbm, pd_hbm, cnt_hbm, out_hbm,
          acc_sp, gathA, gathB, ervA, ervB, isvA, isvB, idvA, idvB,
          ldv, cntv, semHA, semEA, semHB, semEB):
        c = lax.axis_index("c")
        s = lax.axis_index("s")
        lane = lax.iota(jnp.int32, 16)
        zero16 = jnp.zeros((16,), jnp.float32)
        base = pl.multiple_of(s * 784, 16)

        for p in range(2):
            b = 2 * c + p
            lo = b * Q

            @pl.loop(0, CH)
            def _(i):
                for q5 in range(W80 // 16):
                    gathA[i, pl.ds(q5 * 16, 16)] = zero16

            @pl.when(s < 15)
            def _():
                off = 0
                for sz in (CH,) * 6 + (16,):
                    pltpu.sync_copy(gathA.at[pl.ds(0, sz)],
                                    acc_sp.at[pl.ds(base + off, sz)])
                    off += sz

            @pl.when(s == 15)
            def _():
                off = 0
                for sz in (CH,) * 5 + (100,):
                    pltpu.sync_copy(gathA.at[pl.ds(0, sz)],
                                    acc_sp.at[pl.ds(11760 + off, sz)])
                    off += sz

            plsc.subcore_barrier()

            for r in (s, s + 16):
                pltpu.sync_copy(cnt_hbm.at[r, pl.ds(0, 8)],
                                cntv.at[pl.ds(0, 8)])
                cv = cntv[pl.ds(0, 16)]
                n = jnp.sum(jnp.where(lane == b, cv, 0))
                trips = lax.shift_right_logical(n + (CH - 1), 7)

                def fetch(t, iv, dv, gb, ev, sh, se):
                    eoff = pl.multiple_of(t * CH, CH)
                    c1 = pltpu.async_copy(ps_hbm.at[b, r, pl.ds(eoff, CH)],
                                          iv, sh)
                    c2 = pltpu.async_copy(pd_hbm.at[b, r, pl.ds(eoff, CH)],
                                          dv, se)
                    c1.wait()
                    c2.wait()
                    pltpu.async_copy(h_hbm.at[iv], gb, sh)
                    pltpu.async_copy(er_hbm.at[dv], ev, se)

                def drain(iv, dv, gb, ev, sh, se):
                    pltpu.make_async_copy(h_hbm.at[iv], gb, sh).wait()
                    pltpu.make_async_copy(er_hbm.at[dv], ev, se).wait()

                def work(gb, ev, dv):
                    @pl.loop(0, CH, step=16)
                    def _(j):
                        d16 = dv[pl.ds(j, 16)]
                        ldv[pl.ds(j, 16)] = jnp.clip(d16 - lo, 0, Q - 1)

                    @pl.loop(0, CH)
                    def _(i):
                        i16 = jnp.full((16,), i, jnp.int32)
                        er_b = plsc.load_gather(ev, [i16])
                        e = gb[i, pl.ds(64, 16)] + er_b
                        w = jnp.exp(jnp.maximum(e, 0.2 * e))
                        for q5 in range(4):
                            gb[i, pl.ds(q5 * 16, 16)] = (
                                gb[i, pl.ds(q5 * 16, 16)] * w)
                        gb[i, pl.ds(64, 16)] = w

                    pltpu.sync_copy(gb, acc_sp.at[ldv], add=True)

                bufA = (isvA, idvA, gathA, ervA, semHA, semEA)
                bufB = (isvB, idvB, gathB, ervB, semHB, semEB)

                @pl.when(trips > 0)
                def _():
                    fetch(jnp.int32(0), *bufA)

                def pair(u, _):
                    t1 = u * 2 + 1

                    @pl.when(t1 < trips)
                    def _():
                        fetch(t1, *bufB)

                    drain(*bufA)
                    work(gathA, ervA, idvA)

                    @pl.when(t1 + 1 < trips)
                    def _():
                        fetch(t1 + 1, *bufA)

                    @pl.when(t1 < trips)
                    def _():
                        drain(*bufB)
                        work(gathB, ervB, idvB)

                    return 0

                pairs = lax.shift_right_logical(trips + 1, 1)
                lax.fori_loop(0, pairs, pair, jnp.int32(0))

            plsc.subcore_barrier()

            @pl.when(s < 15)
            def _(b=b):
                pltpu.sync_copy(acc_sp.at[pl.ds(base, 784)],
                                out_hbm.at[b, pl.ds(base, 784)])

            @pl.when(s == 15)
            def _(b=b):
                pltpu.sync_copy(acc_sp.at[pl.ds(11760, 740)],
                                out_hbm.at[b, pl.ds(11760, 740)])

    out = k(h80, er1, psrc, pdst, counts)
    return out.reshape(N, W80)


def _proj_from_x(x, wbig, bias, widths):
    n, kdim = x.shape
    m = wbig.shape[1]
    bsz = 1000
    offs = [sum(widths[:t]) for t in range(len(widths))]

    def body(x_ref, w_ref, b_ref, *out_refs):
        y = jnp.dot(x_ref[...], w_ref[...],
                    preferred_element_type=jnp.float32) + b_ref[...]
        for r, o, wd in zip(out_refs, offs, widths):
            r[...] = y[:, o:o + wd]

    return pl.pallas_call(
        body,
        grid=(n // bsz,),
        in_specs=[
            pl.BlockSpec((bsz, kdim), lambda i: (i, 0)),
            pl.BlockSpec((kdim, m), lambda i: (0, 0)),
            pl.BlockSpec((1, m), lambda i: (0, 0)),
        ],
        out_specs=[pl.BlockSpec((bsz, wd), lambda i: (i, 0)) for wd in widths],
        out_shape=[jax.ShapeDtypeStruct((n, wd), jnp.float32) for wd in widths],
    )(x, wbig, bias)


def _proj_from_acc(accs, biases, wcat, widths):
    nacc = len(accs)
    m = wcat.shape[1]
    bsz = 1000
    offs = [sum(widths[:t]) for t in range(len(widths))]

    def body(*refs):
        acc_refs = refs[:nacc]
        b_refs = refs[nacc:2 * nacc]
        w_ref = refs[2 * nacc]
        out_refs = refs[2 * nacc + 1:]
        xs = None
        for a_ref, b_ref in zip(acc_refs, b_refs):
            blk = a_ref[...]
            den = blk[:, 64:65]
            h = jnp.where(den != 0.0, blk[:, :64] / den, 0.0) + b_ref[...]
            xs = h if xs is None else xs + h
        x = jnp.maximum(xs, 0.0)
        y = jnp.dot(x, w_ref[...], preferred_element_type=jnp.float32)
        for r, o, wd in zip(out_refs, offs, widths):
            r[...] = y[:, o:o + wd]

    in_specs = (
        [pl.BlockSpec((bsz, W80), lambda i: (i, 0))] * nacc
        + [pl.BlockSpec((1, 64), lambda i: (0, 0))] * nacc
        + [pl.BlockSpec((64, m), lambda i: (0, 0))]
    )
    return pl.pallas_call(
        body,
        grid=(N // bsz,),
        in_specs=in_specs,
        out_specs=[pl.BlockSpec((bsz, wd), lambda i: (i, 0))
                   for wd in widths],
        out_shape=[jax.ShapeDtypeStruct((N, wd), jnp.float32) for wd in widths],
    )(*accs, *biases, wcat)


def _readout(acc_g, acc_sd, acc_bk, bg, bsd, bbk, w1t, b1, w2t, b2):
    bsz = 1000
    nb = N // bsz

    def body(ag, asd, abk, bg_r, bsd_r, bbk_r, w1, b1_r, w2, b2_r,
             out_ref, accum):
        i = pl.program_id(0)

        @pl.when(i == 0)
        def _():
            accum[...] = jnp.zeros_like(accum)

        def norm(ref):
            blk = ref[...]
            den = blk[:, 64:65]
            return jnp.where(den != 0.0, blk[:, :64] / den, 0.0)

        accum[0:1, :] += jnp.sum(norm(ag), axis=0, keepdims=True)
        accum[1:2, :] += jnp.sum(norm(asd) + norm(abk), axis=0, keepdims=True)

        @pl.when(i == nb - 1)
        def _():
            mean_n = accum[0:1, :] / N + bg_r[...]
            mean_p = accum[1:2, :] / N + bsd_r[...] + bbk_r[...]
            z = 0.5 * (mean_p + mean_n)
            h1 = jnp.maximum(
                jnp.dot(z, w1[...], preferred_element_type=jnp.float32)
                + b1_r[...], 0.0)
            out_ref[...] = (jnp.dot(h1, w2[...],
                                    preferred_element_type=jnp.float32)
                            + b2_r[...])

    full = lambda i: (0, 0)
    return pl.pallas_call(
        body,
        grid=(nb,),
        in_specs=(
            [pl.BlockSpec((bsz, W80), lambda i: (i, 0))] * 3
            + [pl.BlockSpec((1, 64), full)] * 3
            + [pl.BlockSpec((64, 64), full), pl.BlockSpec((1, 64), full)] * 2
        ),
        out_specs=pl.BlockSpec((1, 64), full),
        out_shape=jax.ShapeDtypeStruct((1, 64), jnp.float32),
        scratch_shapes=[pltpu.VMEM((8, 64), jnp.float32)],
    )(acc_g, acc_sd, acc_bk, bg, bsd, bbk, w1t, b1, w2t, b2)


def _rep16(v):
    return jnp.tile(v[:, None], (1, 16))


def _src_table_w(gp):
    wt = gp["W"].T
    return jnp.concatenate([wt, _rep16(wt @ gp["attn_l"])], axis=1)


def _er_w(gp):
    return _rep16(gp["W"].T @ gp["attn_r"])


def _pad_er(er16):
    return jnp.concatenate([er16[:, 0], jnp.full((1,), NEG, jnp.float32)])


def _prep_edges(eidx):
    es = eidx[0].astype(jnp.int32)
    ed = eidx[1].astype(jnp.int32)
    pad = EPAD - E
    es = jnp.concatenate([es, jnp.zeros((pad,), jnp.int32)])
    ed = jnp.concatenate([ed, jnp.full((pad,), N, jnp.int32)])
    return es, ed


def kernel(x_pmos, x_nmos, params, eidx_gate, eidx_sd, eidx_back):
    p = params
    esg, edg = _prep_edges(eidx_gate)
    ess, eds = _prep_edges(eidx_sd)
    esb, edb = _prep_edges(eidx_back)

    part_g = _partition_edges(esg, edg)
    part_s = _partition_edges(ess, eds)
    part_b = _partition_edges(esb, edb)

    wcat_p1 = jnp.concatenate(
        [_src_table_w(p["l1_gate"]), _src_table_w(p["l1_back"]),
         _er_w(p["l1_sd"]), _er_w(p["l1_back"])], axis=1)
    wcat_n1 = jnp.concatenate(
        [_src_table_w(p["l1_sd"]), _er_w(p["l1_gate"])], axis=1)
    wbig_p1 = p["We_p"].T @ wcat_p1
    bias_p1 = (p["be_p"] @ wcat_p1)[None, :]
    wbig_n1 = p["We_n"].T @ wcat_n1
    bias_n1 = (p["be_n"] @ wcat_n1)[None, :]

    hg80, hb80, er_sd, er_bk = _proj_from_x(
        x_pmos, wbig_p1, bias_p1, [80, 80, 16, 16])
    hs80, er_g = _proj_from_x(x_nmos, wbig_n1, bias_n1, [80, 16])

    acc_g1 = _gat_edges(hg80, _pad_er(er_g), *part_g)
    acc_s1 = _gat_edges(hs80, _pad_er(er_sd), *part_s)
    acc_b1 = _gat_edges(hb80, _pad_er(er_bk), *part_b)

    wcat_p2 = jnp.concatenate(
        [_src_table_w(p["l2_gate"]), _src_table_w(p["l2_back"]),
         _er_w(p["l2_sd"]), _er_w(p["l2_back"])], axis=1)
    wcat_n2 = jnp.concatenate(
        [_src_table_w(p["l2_sd"]), _er_w(p["l2_gate"])], axis=1)

    b_g1 = p["l1_gate"]["b"][None, :]
    b_s1 = p["l1_sd"]["b"][None, :]
    b_b1 = p["l1_back"]["b"][None, :]

    hg80_2, hb80_2, er_sd2, er_bk2 = _proj_from_acc(
        [acc_s1, acc_b1], [b_s1, b_b1], wcat_p2, [80, 80, 16, 16])
    hs80_2, er_g2 = _proj_from_acc([acc_g1], [b_g1], wcat_n2, [80, 16])

    acc_g2 = _gat_edges(hg80_2, _pad_er(er_g2), *part_g)
    acc_s2 = _gat_edges(hs80_2, _pad_er(er_sd2), *part_s)
    acc_b2 = _gat_edges(hb80_2, _pad_er(er_bk2), *part_b)

    y = _readout(
        acc_g2, acc_s2, acc_b2,
        p["l2_gate"]["b"][None, :], p["l2_sd"]["b"][None, :],
        p["l2_back"]["b"][None, :],
        p["Wr1"].T, p["br1"][None, :], p["Wr2"].T, p["br2"][None, :])
    return y.reshape(-1)


# speedup vs baseline: 35.3460x; 1.2364x over previous
import functools

import jax
import jax.numpy as jnp
from jax import lax
from jax.experimental import pallas as pl
from jax.experimental.pallas import tpu as pltpu
from jax.experimental.pallas import tpu_sc as plsc

N = 50000
Q = 12500
E = 800000
EPAD = 800768
EPW = 25024
CHP = 64
NCHP = 391
REG = 25216
CH = 128
W80 = 80
NEG = -1e30

_MESH = dict(core_axis_name="c", subcore_axis_name="s")
_CP = dict(use_tc_tiling_on_sc=False, needs_layout_passes=False)


def _partition_edges(esrc, edst):
    @functools.partial(
        pl.kernel,
        out_type=(jax.ShapeDtypeStruct((4, 32, REG), jnp.int32),
                  jax.ShapeDtypeStruct((4, 32, REG), jnp.int32),
                  jax.ShapeDtypeStruct((32, 8), jnp.int32)),
        mesh=plsc.VectorSubcoreMesh(**_MESH),
        compiler_params=pltpu.CompilerParams(**_CP),
        scratch_types=[
            pltpu.VMEM((CHP,), jnp.int32),
            pltpu.VMEM((CHP,), jnp.int32),
            pltpu.VMEM((CHP,), jnp.int32),
            pltpu.VMEM((CHP,), jnp.int32),
            pltpu.VMEM((4, 192), jnp.int32),
            pltpu.VMEM((4, 192), jnp.int32),
            pltpu.VMEM((16,), jnp.int32),
            pltpu.SemaphoreType.DMA,
            pltpu.SemaphoreType.DMA,
            pltpu.SemaphoreType.DMA,
            pltpu.SemaphoreType.DMA,
        ],
    )
    def k(es_hbm, ed_hbm, ps_hbm, pd_hbm, cnt_hbm,
          isvA, idvA, isvB, idvB, stg_s, stg_d, cntv,
          semSA, semDA, semSB, semDB):
        c = lax.axis_index("c")
        s = lax.axis_index("s")
        wid = c * 16 + s
        ebase = wid * EPW
        lane = lax.iota(jnp.int32, 16)

        def issue(t, iv, dv, s1, s2):
            eoff = ebase + t * CHP
            pltpu.async_copy(es_hbm.at[pl.ds(eoff, CHP)], iv, s1)
            pltpu.async_copy(ed_hbm.at[pl.ds(eoff, CHP)], dv, s2)

        def wait_idx(t, iv, dv, s1, s2):
            eoff = ebase + t * CHP
            pltpu.make_async_copy(es_hbm.at[pl.ds(eoff, CHP)], iv, s1).wait()
            pltpu.make_async_copy(ed_hbm.at[pl.ds(eoff, CHP)], dv, s2).wait()

        def process(isv, idv, carry):
            curs = list(carry[:4])
            outs = list(carry[4:])
            for g in range(CHP // 16):
                d16 = idv[pl.ds(g * 16, 16)]
                s16 = isv[pl.ds(g * 16, 16)]
                masks = [d16 < Q,
                         (d16 >= Q) & (d16 < 2 * Q),
                         (d16 >= 2 * Q) & (d16 < 3 * Q),
                         d16 >= 3 * Q]
                for b in range(4):
                    mb = masks[b]
                    plsc.store_compressed(stg_s.at[b, pl.ds(curs[b], 16)],
                                          s16, mask=mb)
                    plsc.store_compressed(stg_d.at[b, pl.ds(curs[b], 16)],
                                          d16, mask=mb)
                    newc = curs[b] + jnp.sum(jnp.where(mb, 1, 0))
                    fl = newc >= 64

                    @pl.when(fl)
                    def _(b=b, ob=pl.multiple_of(outs[b], 64)):
                        pltpu.sync_copy(stg_s.at[b, pl.ds(0, 64)],
                                        ps_hbm.at[b, wid, pl.ds(ob, 64)])
                        pltpu.sync_copy(stg_d.at[b, pl.ds(0, 64)],
                                        pd_hbm.at[b, wid, pl.ds(ob, 64)])
                        stg_s[b, pl.ds(0, 16)] = stg_s[b, pl.ds(64, 16)]
                        stg_d[b, pl.ds(0, 16)] = stg_d[b, pl.ds(64, 16)]

                    curs[b] = newc - jnp.where(fl, 64, 0)
                    outs[b] = outs[b] + jnp.where(fl, 64, 0)
            return (*curs, *outs)

        def pair_body(u, carry):
            t0 = u * 2
            issue(t0 + 1, isvB, idvB, semSB, semDB)
            wait_idx(t0, isvA, idvA, semSA, semDA)
            carry = process(isvA, idvA, carry)
            issue(t0 + 2, isvA, idvA, semSA, semDA)
            wait_idx(t0 + 1, isvB, idvB, semSB, semDB)
            carry = process(isvB, idvB, carry)
            return carry

        zero = jnp.int32(0)
        issue(jnp.int32(0), isvA, idvA, semSA, semDA)
        carry = lax.fori_loop(0, (NCHP - 1) // 2, pair_body, (zero,) * 8)
        wait_idx(jnp.int32(NCHP - 1), isvA, idvA, semSA, semDA)
        carry = process(isvA, idvA, carry)
        curs = carry[:4]
        outs = [pl.multiple_of(o, 64) for o in carry[4:]]
        pad_s = jnp.zeros((16,), jnp.int32)
        pad_d = jnp.full((16,), N, jnp.int32)
        cnts = []
        for b in range(4):
            for kk in range(4):
                stg_s[b, pl.ds(curs[b] + kk * 16, 16)] = pad_s
                stg_d[b, pl.ds(curs[b] + kk * 16, 16)] = pad_d
            pltpu.sync_copy(stg_s.at[b, pl.ds(0, 64)],
                            ps_hbm.at[b, wid, pl.ds(outs[b], 64)])
            pltpu.sync_copy(stg_d.at[b, pl.ds(0, 64)],
                            pd_hbm.at[b, wid, pl.ds(outs[b], 64)])
            for kk in range(8):
                stg_s[b, pl.ds(kk * 16, 16)] = pad_s
                stg_d[b, pl.ds(kk * 16, 16)] = pad_d
            pltpu.sync_copy(stg_s.at[b, pl.ds(0, 64)],
                            ps_hbm.at[b, wid, pl.ds(outs[b] + 64, 64)])
            pltpu.sync_copy(stg_d.at[b, pl.ds(0, 64)],
                            pd_hbm.at[b, wid, pl.ds(outs[b] + 64, 64)])
            pltpu.sync_copy(stg_s.at[b, pl.ds(64, 64)],
                            ps_hbm.at[b, wid, pl.ds(outs[b] + 128, 64)])
            pltpu.sync_copy(stg_d.at[b, pl.ds(64, 64)],
                            pd_hbm.at[b, wid, pl.ds(outs[b] + 128, 64)])
            cnts.append(outs[b] + curs[b])
        cv = jnp.where(lane == 0, cnts[0],
                       jnp.where(lane == 1, cnts[1],
                                 jnp.where(lane == 2, cnts[2],
                                           jnp.where(lane == 3, cnts[3], 0))))
        cntv[pl.ds(0, 16)] = cv
        pltpu.sync_copy(cntv.at[pl.ds(0, 8)], cnt_hbm.at[wid, pl.ds(0, 8)])

    return k(esrc, edst)


def _gat_edges(h80, er1, psrc, pdst, counts):
    @functools.partial(
        pl.kernel,
        out_type=jax.ShapeDtypeStruct((4, Q, W80), jnp.float32),
        mesh=plsc.VectorSubcoreMesh(**_MESH),
        compiler_params=pltpu.CompilerParams(**_CP),
        scratch_types=[
            pltpu.VMEM_SHARED((Q, W80), jnp.float32),
            pltpu.VMEM((CH, W80), jnp.float32),
            pltpu.VMEM((CH, W80), jnp.float32),
            pltpu.VMEM((CH,), jnp.float32),
            pltpu.VMEM((CH,), jnp.float32),
            pltpu.VMEM((CH,), jnp.int32),
            pltpu.VMEM((CH,), jnp.int32),
            pltpu.VMEM((CH,), jnp.int32),
            pltpu.VMEM((CH,), jnp.int32),
            pltpu.VMEM((CH,), jnp.int32),
            pltpu.VMEM((CH,), jnp.float32),
            pltpu.VMEM((16,), jnp.int32),
            pltpu.SemaphoreType.DMA,
            pltpu.SemaphoreType.DMA,
            pltpu.SemaphoreType.DMA,
            pltpu.SemaphoreType.DMA,
        ],
    )
    def k(h_hbm, er_hbm, ps_hbm, pd_hbm, cnt_hbm, out_hbm,
          acc_sp, gathA, gathB, ervA, ervB, isvA, isvB, idvA, idvB,
          ldv, wv, cntv, semHA, semEA, semHB, semEB):
        c = lax.axis_index("c")
        s = lax.axis_index("s")
        lane = lax.iota(jnp.int32, 16)
        zero16 = jnp.zeros((16,), jnp.float32)
        base = pl.multiple_of(s * 784, 16)

        for p in range(2):
            b = 2 * c + p
            lo = b * Q

            @pl.loop(0, CH)
            def _(i):
                for q5 in range(W80 // 16):
                    gathA[i, pl.ds(q5 * 16, 16)] = zero16

            @pl.when(s < 15)
            def _():
                off = 0
                for sz in (CH,) * 6 + (16,):
                    pltpu.sync_copy(gathA.at[pl.ds(0, sz)],
                                    acc_sp.at[pl.ds(base + off, sz)])
                    off += sz

            @pl.when(s == 15)
            def _():
                off = 0
                for sz in (CH,) * 5 + (100,):
                    pltpu.sync_copy(gathA.at[pl.ds(0, sz)],
                                    acc_sp.at[pl.ds(11760 + off, sz)])
                    off += sz

            plsc.subcore_barrier()

            for r in (s, s + 16):
                pltpu.sync_copy(cnt_hbm.at[r, pl.ds(0, 8)],
                                cntv.at[pl.ds(0, 8)])
                cv = cntv[pl.ds(0, 16)]
                n = jnp.sum(jnp.where(lane == b, cv, 0))
                trips = lax.shift_right_logical(n + (CH - 1), 7)

                def fetch(t, iv, dv, gb, ev, sh, se):
                    eoff = pl.multiple_of(t * CH, CH)
                    c1 = pltpu.async_copy(ps_hbm.at[b, r, pl.ds(eoff, CH)],
                                          iv, sh)
                    c2 = pltpu.async_copy(pd_hbm.at[b, r, pl.ds(eoff, CH)],
                                          dv, se)
                    c1.wait()
                    c2.wait()
                    pltpu.async_copy(h_hbm.at[iv], gb, sh)
                    pltpu.async_copy(er_hbm.at[dv], ev, se)

                def drain(iv, dv, gb, ev, sh, se):
                    pltpu.make_async_copy(h_hbm.at[iv], gb, sh).wait()
                    pltpu.make_async_copy(er_hbm.at[dv], ev, se).wait()

                def work(gb, ev, dv):
                    col64 = jnp.full((16,), 64, jnp.int32)

                    @pl.loop(0, CH, step=16)
                    def _(j):
                        d16 = dv[pl.ds(j, 16)]
                        ldv[pl.ds(j, 16)] = jnp.clip(d16 - lo, 0, Q - 1)
                        row16 = lax.iota(jnp.int32, 16) + j
                        el16 = plsc.load_gather(gb, [row16, col64])
                        e16 = el16 + ev[pl.ds(j, 16)]
                        wv[pl.ds(j, 16)] = jnp.exp(jnp.maximum(e16, 0.2 * e16))

                    @pl.loop(0, CH, step=4)
                    def _(i):
                        for t in range(4):
                            ii = i + t
                            w = plsc.load_gather(
                                wv, [jnp.full((16,), ii, jnp.int32)])
                            for q5 in range(4):
                                gb[ii, pl.ds(q5 * 16, 16)] = (
                                    gb[ii, pl.ds(q5 * 16, 16)] * w)
                            gb[ii, pl.ds(64, 16)] = w

                    pltpu.sync_copy(gb, acc_sp.at[ldv], add=True)

                bufA = (isvA, idvA, gathA, ervA, semHA, semEA)
                bufB = (isvB, idvB, gathB, ervB, semHB, semEB)

                @pl.when(trips > 0)
                def _():
                    fetch(jnp.int32(0), *bufA)

                def pair(u, _):
                    t1 = u * 2 + 1

                    @pl.when(t1 < trips)
                    def _():
                        fetch(t1, *bufB)

                    drain(*bufA)
                    work(gathA, ervA, idvA)

                    @pl.when(t1 + 1 < trips)
                    def _():
                        fetch(t1 + 1, *bufA)

                    @pl.when(t1 < trips)
                    def _():
                        drain(*bufB)
                        work(gathB, ervB, idvB)

                    return 0

                pairs = lax.shift_right_logical(trips + 1, 1)
                lax.fori_loop(0, pairs, pair, jnp.int32(0))

            plsc.subcore_barrier()

            @pl.when(s < 15)
            def _(b=b):
                pltpu.sync_copy(acc_sp.at[pl.ds(base, 784)],
                                out_hbm.at[b, pl.ds(base, 784)])

            @pl.when(s == 15)
            def _(b=b):
                pltpu.sync_copy(acc_sp.at[pl.ds(11760, 740)],
                                out_hbm.at[b, pl.ds(11760, 740)])

    out = k(h80, er1, psrc, pdst, counts)
    return out.reshape(N, W80)


def _proj_from_x(x, wbig, bias, widths):
    n, kdim = x.shape
    m = wbig.shape[1]
    bsz = 1000
    offs = [sum(widths[:t]) for t in range(len(widths))]

    def body(x_ref, w_ref, b_ref, *out_refs):
        y = jnp.dot(x_ref[...], w_ref[...],
                    preferred_element_type=jnp.float32) + b_ref[...]
        for r, o, wd in zip(out_refs, offs, widths):
            r[...] = y[:, o:o + wd]

    return pl.pallas_call(
        body,
        grid=(n // bsz,),
        in_specs=[
            pl.BlockSpec((bsz, kdim), lambda i: (i, 0)),
            pl.BlockSpec((kdim, m), lambda i: (0, 0)),
            pl.BlockSpec((1, m), lambda i: (0, 0)),
        ],
        out_specs=[pl.BlockSpec((bsz, wd), lambda i: (i, 0)) for wd in widths],
        out_shape=[jax.ShapeDtypeStruct((n, wd), jnp.float32) for wd in widths],
    )(x, wbig, bias)


def _proj_from_acc(accs, biases, wcat, widths):
    nacc = len(accs)
    m = wcat.shape[1]
    bsz = 1000
    offs = [sum(widths[:t]) for t in range(len(widths))]

    def body(*refs):
        acc_refs = refs[:nacc]
        b_refs = refs[nacc:2 * nacc]
        w_ref = refs[2 * nacc]
        out_refs = refs[2 * nacc + 1:]
        xs = None
        for a_ref, b_ref in zip(acc_refs, b_refs):
            blk = a_ref[...]
            den = blk[:, 64:65]
            h = jnp.where(den != 0.0, blk[:, :64] / den, 0.0) + b_ref[...]
            xs = h if xs is None else xs + h
        x = jnp.maximum(xs, 0.0)
        y = jnp.dot(x, w_ref[...], preferred_element_type=jnp.float32)
        for r, o, wd in zip(out_refs, offs, widths):
            r[...] = y[:, o:o + wd]

    in_specs = (
        [pl.BlockSpec((bsz, W80), lambda i: (i, 0))] * nacc
        + [pl.BlockSpec((1, 64), lambda i: (0, 0))] * nacc
        + [pl.BlockSpec((64, m), lambda i: (0, 0))]
    )
    return pl.pallas_call(
        body,
        grid=(N // bsz,),
        in_specs=in_specs,
        out_specs=[pl.BlockSpec((bsz, wd), lambda i: (i, 0))
                   for wd in widths],
        out_shape=[jax.ShapeDtypeStruct((N, wd), jnp.float32) for wd in widths],
    )(*accs, *biases, wcat)


def _readout(acc_g, acc_sd, acc_bk, bg, bsd, bbk, w1t, b1, w2t, b2):
    bsz = 1000
    nb = N // bsz

    def body(ag, asd, abk, bg_r, bsd_r, bbk_r, w1, b1_r, w2, b2_r,
             out_ref, accum):
        i = pl.program_id(0)

        @pl.when(i == 0)
        def _():
            accum[...] = jnp.zeros_like(accum)

        def norm(ref):
            blk = ref[...]
            den = blk[:, 64:65]
            return jnp.where(den != 0.0, blk[:, :64] / den, 0.0)

        accum[0:1, :] += jnp.sum(norm(ag), axis=0, keepdims=True)
        accum[1:2, :] += jnp.sum(norm(asd) + norm(abk), axis=0, keepdims=True)

        @pl.when(i == nb - 1)
        def _():
            mean_n = accum[0:1, :] / N + bg_r[...]
            mean_p = accum[1:2, :] / N + bsd_r[...] + bbk_r[...]
            z = 0.5 * (mean_p + mean_n)
            h1 = jnp.maximum(
                jnp.dot(z, w1[...], preferred_element_type=jnp.float32)
                + b1_r[...], 0.0)
            out_ref[...] = (jnp.dot(h1, w2[...],
                                    preferred_element_type=jnp.float32)
                            + b2_r[...])

    full = lambda i: (0, 0)
    return pl.pallas_call(
        body,
        grid=(nb,),
        in_specs=(
            [pl.BlockSpec((bsz, W80), lambda i: (i, 0))] * 3
            + [pl.BlockSpec((1, 64), full)] * 3
            + [pl.BlockSpec((64, 64), full), pl.BlockSpec((1, 64), full)] * 2
        ),
        out_specs=pl.BlockSpec((1, 64), full),
        out_shape=jax.ShapeDtypeStruct((1, 64), jnp.float32),
        scratch_shapes=[pltpu.VMEM((8, 64), jnp.float32)],
    )(acc_g, acc_sd, acc_bk, bg, bsd, bbk, w1t, b1, w2t, b2)


def _rep16(v):
    return jnp.tile(v[:, None], (1, 16))


def _src_table_w(gp):
    wt = gp["W"].T
    return jnp.concatenate([wt, _rep16(wt @ gp["attn_l"])], axis=1)


def _er_w(gp):
    return _rep16(gp["W"].T @ gp["attn_r"])


def _pad_er(er16):
    return jnp.concatenate([er16[:, 0], jnp.full((1,), NEG, jnp.float32)])


def _prep_edges(eidx):
    es = eidx[0].astype(jnp.int32)
    ed = eidx[1].astype(jnp.int32)
    pad = EPAD - E
    es = jnp.concatenate([es, jnp.zeros((pad,), jnp.int32)])
    ed = jnp.concatenate([ed, jnp.full((pad,), N, jnp.int32)])
    return es, ed


def kernel(x_pmos, x_nmos, params, eidx_gate, eidx_sd, eidx_back):
    p = params
    esg, edg = _prep_edges(eidx_gate)
    ess, eds = _prep_edges(eidx_sd)
    esb, edb = _prep_edges(eidx_back)

    part_g = _partition_edges(esg, edg)
    part_s = _partition_edges(ess, eds)
    part_b = _partition_edges(esb, edb)

    wcat_p1 = jnp.concatenate(
        [_src_table_w(p["l1_gate"]), _src_table_w(p["l1_back"]),
         _er_w(p["l1_sd"]), _er_w(p["l1_back"])], axis=1)
    wcat_n1 = jnp.concatenate(
        [_src_table_w(p["l1_sd"]), _er_w(p["l1_gate"])], axis=1)
    wbig_p1 = p["We_p"].T @ wcat_p1
    bias_p1 = (p["be_p"] @ wcat_p1)[None, :]
    wbig_n1 = p["We_n"].T @ wcat_n1
    bias_n1 = (p["be_n"] @ wcat_n1)[None, :]

    hg80, hb80, er_sd, er_bk = _proj_from_x(
        x_pmos, wbig_p1, bias_p1, [80, 80, 16, 16])
    hs80, er_g = _proj_from_x(x_nmos, wbig_n1, bias_n1, [80, 16])

    acc_g1 = _gat_edges(hg80, _pad_er(er_g), *part_g)
    acc_s1 = _gat_edges(hs80, _pad_er(er_sd), *part_s)
    acc_b1 = _gat_edges(hb80, _pad_er(er_bk), *part_b)

    wcat_p2 = jnp.concatenate(
        [_src_table_w(p["l2_gate"]), _src_table_w(p["l2_back"]),
         _er_w(p["l2_sd"]), _er_w(p["l2_back"])], axis=1)
    wcat_n2 = jnp.concatenate(
        [_src_table_w(p["l2_sd"]), _er_w(p["l2_gate"])], axis=1)

    b_g1 = p["l1_gate"]["b"][None, :]
    b_s1 = p["l1_sd"]["b"][None, :]
    b_b1 = p["l1_back"]["b"][None, :]

    hg80_2, hb80_2, er_sd2, er_bk2 = _proj_from_acc(
        [acc_s1, acc_b1], [b_s1, b_b1], wcat_p2, [80, 80, 16, 16])
    hs80_2, er_g2 = _proj_from_acc([acc_g1], [b_g1], wcat_n2, [80, 16])

    acc_g2 = _gat_edges(hg80_2, _pad_er(er_g2), *part_g)
    acc_s2 = _gat_edges(hs80_2, _pad_er(er_sd2), *part_s)
    acc_b2 = _gat_edges(hb80_2, _pad_er(er_bk2), *part_b)

    y = _readout(
        acc_g2, acc_s2, acc_b2,
        p["l2_gate"]["b"][None, :], p["l2_sd"]["b"][None, :],
        p["l2_back"]["b"][None, :],
        p["Wr1"].T, p["br1"][None, :], p["Wr2"].T, p["br2"][None, :])
    return y.reshape(-1)
